# Initial kernel scaffold; baseline (speedup 1.0000x reference)
import jax
import jax.numpy as jnp
from jax.experimental import pallas as pl


def kernel(logits, labels):
    raise NotImplementedError("write your pallas kernel here")


# speedup vs baseline: 1.0962x; 1.0962x over previous
import jax
import jax.numpy as jnp
from jax.experimental import pallas as pl

B, C = 1024, 100000
S = 64.0
M3 = 0.4
ROW_BLK = 8


def _margin_scale_kernel(labels_ref, logits_ref, out_ref):
    x = logits_ref[...]
    lab = labels_ref[...]
    cols = jax.lax.broadcasted_iota(jnp.int32, x.shape, dimension=1)
    hit = cols == lab
    out_ref[...] = x * S - jnp.where(hit, M3 * S, 0.0)


def kernel(logits, labels):
    valid = labels != -1
    safe_labels = jnp.where(valid, labels, -2)
    lab2d = safe_labels.reshape(B, 1)
    grid = (B // ROW_BLK,)
    return pl.pallas_call(
        _margin_scale_kernel,
        grid=grid,
        in_specs=[
            pl.BlockSpec((ROW_BLK, 1), lambda i: (i, 0)),
            pl.BlockSpec((ROW_BLK, C), lambda i: (i, 0)),
        ],
        out_specs=pl.BlockSpec((ROW_BLK, C), lambda i: (i, 0)),
        out_shape=jax.ShapeDtypeStruct((B, C), jnp.float32),
    )(lab2d, logits)


# speedup vs baseline: 1.1163x; 1.0183x over previous
import jax
import jax.numpy as jnp
from jax.experimental import pallas as pl

B, C = 1024, 100000
S = 64.0
M3 = 0.4
ROW_BLK = 16


def _margin_scale_kernel(labels_ref, logits_ref, out_ref):
    x = logits_ref[...]
    lab = labels_ref[...]
    cols = jax.lax.broadcasted_iota(jnp.int32, x.shape, dimension=1)
    hit = cols == lab
    out_ref[...] = x * S - jnp.where(hit, M3 * S, 0.0)


def kernel(logits, labels):
    valid = labels != -1
    safe_labels = jnp.where(valid, labels, -2)
    lab2d = safe_labels.reshape(B, 1)
    grid = (B // ROW_BLK,)
    return pl.pallas_call(
        _margin_scale_kernel,
        grid=grid,
        in_specs=[
            pl.BlockSpec((ROW_BLK, 1), lambda i: (i, 0)),
            pl.BlockSpec((ROW_BLK, C), lambda i: (i, 0)),
        ],
        out_specs=pl.BlockSpec((ROW_BLK, C), lambda i: (i, 0)),
        out_shape=jax.ShapeDtypeStruct((B, C), jnp.float32),
    )(lab2d, logits)


# speedup vs baseline: 1.1195x; 1.0028x over previous
import jax
import jax.numpy as jnp
from jax.experimental import pallas as pl

B, C = 1024, 100000
S = 64.0
M3 = 0.4
ROW_BLK = 32


def _margin_scale_kernel(labels_ref, margins_ref, logits_ref, out_ref):
    out_ref[...] = logits_ref[...] * S
    idx = jax.lax.broadcasted_iota(jnp.int32, (1, 128), 1)
    for r in range(ROW_BLK):
        l = labels_ref[r, 0]
        m = margins_ref[r, 0]
        base = (l // 128) * 128
        off = l - base
        chunk = logits_ref[r : r + 1, pl.ds(base, 128)]
        out_ref[r : r + 1, pl.ds(base, 128)] = (
            chunk - jnp.where(idx == off, m, 0.0)
        ) * S


def kernel(logits, labels):
    valid = labels != -1
    lab2d = jnp.where(valid, labels, 0).reshape(B, 1)
    mar2d = jnp.where(valid, M3, 0.0).astype(jnp.float32).reshape(B, 1)
    grid = (B // ROW_BLK,)
    out = pl.pallas_call(
        _margin_scale_kernel,
        grid=grid,
        in_specs=[
            pl.BlockSpec((ROW_BLK, 1), lambda i: (i, 0)),
            pl.BlockSpec((ROW_BLK, 1), lambda i: (i, 0)),
            pl.BlockSpec((ROW_BLK, C), lambda i: (i, 0)),
        ],
        out_specs=pl.BlockSpec((ROW_BLK, C), lambda i: (i, 0)),
        out_shape=jax.ShapeDtypeStruct((B, C), jnp.float32),
    )(lab2d, mar2d, logits)
    return out
